# baseline (speedup 1.0000x reference)
import functools

import jax
import jax.numpy as jnp
from jax import lax
from jax.experimental import pallas as pl
from jax.experimental.pallas import tpu as pltpu
from jax.experimental.pallas import tpu_sc as plsc

VOCAB = 100000
D = 128
B = 1024
S = 200

NC = 2
NS = 16
NBG = 8
NSG = 4
BCH = B // NBG
SW = S // NSG
SAL = 48
PW = 56
NBUF = 3
NV = D // 16
IDRAIN = 8


def _worker_body(inp_hbm, emb_hbm, pos_hbm, out_hbm,
                 aff_v, idx_v, pos_idx_v, pos_v, gbufs, obufs, oaffs, isem_a,
                 isem_b, gsems, ssems, psem):
    wid = lax.axis_index("s") * NC + lax.axis_index("c")
    bg = wid // NSG
    sg = wid - bg * NSG
    b0 = bg * BCH
    s0 = sg * SW
    align = sg * SAL

    lanes_s = lax.iota(jnp.int32, 16) * S
    base = b0 * S + s0

    @plsc.parallel_loop(0, SW, unroll=2)
    def _(j):
        for g in range(NV):
            aff_v[pl.ds(pl.multiple_of(j * BCH + g * 16, 16), 16)] = (
                lanes_s + (base + g * 16 * S + j))
        src = inp_hbm.at[aff_v.at[pl.ds(pl.multiple_of(j * BCH, BCH), BCH)]]
        dst = idx_v.at[pl.ds(pl.multiple_of(j * BCH, BCH), BCH)]

        @pl.when(j < IDRAIN)
        def _():
            pltpu.async_copy(src, dst, isem_a)

        @pl.when(j >= IDRAIN)
        def _():
            pltpu.async_copy(src, dst, isem_b)

    pltpu.sync_copy(inp_hbm.at[pl.ds(0, S)], pos_idx_v)
    pltpu.async_copy(
        pos_hbm.at[pos_idx_v.at[pl.ds(pl.multiple_of(align, 8), PW)]],
        pos_v, psem)

    pltpu.make_async_copy(inp_hbm.at[pl.ds(0, IDRAIN * BCH)],
                          idx_v.at[pl.ds(0, IDRAIN * BCH)], isem_a).wait()

    def start_gather(j, b):
        idx_slice = idx_v.at[pl.ds(pl.multiple_of(j * BCH, BCH), BCH)]
        pltpu.async_copy(emb_hbm.at[idx_slice], gbufs[b], gsems[b])

    for b in range(NBUF):
        start_gather(b, b)

    pltpu.make_async_copy(pos_hbm.at[pl.ds(0, PW)], pos_v, psem).wait()

    def do_chunk(j, b):
        @pl.when(j == IDRAIN - NBUF)
        def _():
            pltpu.make_async_copy(
                inp_hbm.at[pl.ds(0, (SW - IDRAIN) * BCH)],
                idx_v.at[pl.ds(0, (SW - IDRAIN) * BCH)], isem_b).wait()

        pltpu.make_async_copy(emb_hbm.at[pl.ds(0, BCH)], gbufs[b],
                              gsems[b]).wait()

        @pl.when(j >= NBUF)
        def _():
            pltpu.make_async_copy(
                gbufs[b], out_hbm.at[pl.ds(0, BCH)], ssems[b]).wait()

        pltpu.async_copy(gbufs[b], out_hbm.at[pl.ds(0, BCH)], ssems[b])

        @pl.when(j + NBUF < SW)
        def _():
            start_gather(j + NBUF, b)

    def outer(i, _):
        jo = i * NBUF
        for b in range(NBUF):
            do_chunk(jo + b, b)
        return 0

    nfull = SW // NBUF
    lax.fori_loop(0, nfull, outer, 0)
    for j in range(nfull * NBUF, SW):
        do_chunk(j, j % NBUF)

    for b in range(NBUF):
        pltpu.make_async_copy(gbufs[b], out_hbm.at[pl.ds(0, BCH)],
                              ssems[b]).wait()


@functools.partial(jax.jit, static_argnums=())
def kernel(inp, embed_table, pos_table):
    mesh = plsc.VectorSubcoreMesh(core_axis_name="c", subcore_axis_name="s")
    scratch = (
        [pltpu.VMEM((SW * BCH,), jnp.int32),
         pltpu.VMEM((SW * BCH,), jnp.int32),
         pltpu.VMEM((S,), jnp.int32),
         pltpu.VMEM((PW, D), jnp.float32)]
        + [[pltpu.VMEM((BCH, D), jnp.float32) for _ in range(NBUF)]]
        + [[pltpu.VMEM((BCH, D), jnp.float32) for _ in range(NBUF)]]
        + [[pltpu.VMEM((BCH,), jnp.int32) for _ in range(NBUF)]]
        + [pltpu.SemaphoreType.DMA]
        + [pltpu.SemaphoreType.DMA]
        + [[pltpu.SemaphoreType.DMA for _ in range(NBUF)]]
        + [[pltpu.SemaphoreType.DMA for _ in range(NBUF)]]
        + [pltpu.SemaphoreType.DMA]
    )
    run = pl.kernel(
        _worker_body,
        out_type=jax.ShapeDtypeStruct((B * S, D), jnp.float32),
        mesh=mesh,
        scratch_types=scratch,
    )
    inp = inp.astype(jnp.int32)
    out = run(inp.reshape(B * S), embed_table, pos_table)
    return out.reshape(B, S, D)


# speedup vs baseline: 2.5342x; 2.5342x over previous
import functools

import jax
import jax.numpy as jnp
from jax import lax
from jax.experimental import pallas as pl
from jax.experimental.pallas import tpu as pltpu
from jax.experimental.pallas import tpu_sc as plsc

VOCAB = 100000
D = 128
B = 1024
S = 200

NC = 2
NS = 16
NBG = 8
NSG = 4
BCH = B // NBG
SW = S // NSG
SAL = 48
PW = 56
NBUF = 3
NV = D // 16
IDRAIN = 8


def _worker_body(inp_hbm, emb_hbm, pos_hbm, out_hbm,
                 aff_v, idx_v, pos_idx_v, pos_v, gbufs, obufs, oaffs, isem_a,
                 isem_b, gsems, ssems, psem):
    wid = lax.axis_index("s") * NC + lax.axis_index("c")
    bg = wid // NSG
    sg = wid - bg * NSG
    b0 = bg * BCH
    s0 = sg * SW
    align = sg * SAL

    lanes_s = lax.iota(jnp.int32, 16) * S
    base = b0 * S + s0

    @plsc.parallel_loop(0, SW, unroll=2)
    def _(j):
        for g in range(NV):
            aff_v[pl.ds(pl.multiple_of(j * BCH + g * 16, 16), 16)] = (
                lanes_s + (base + g * 16 * S + j))
        src = inp_hbm.at[aff_v.at[pl.ds(pl.multiple_of(j * BCH, BCH), BCH)]]
        dst = idx_v.at[pl.ds(pl.multiple_of(j * BCH, BCH), BCH)]

        @pl.when(j < IDRAIN)
        def _():
            pltpu.async_copy(src, dst, isem_a)

        @pl.when(j >= IDRAIN)
        def _():
            pltpu.async_copy(src, dst, isem_b)

    pltpu.sync_copy(inp_hbm.at[pl.ds(0, S)], pos_idx_v)
    pltpu.async_copy(
        pos_hbm.at[pos_idx_v.at[pl.ds(pl.multiple_of(align, 8), PW)]],
        pos_v, psem)

    pltpu.make_async_copy(inp_hbm.at[pl.ds(0, IDRAIN * BCH)],
                          idx_v.at[pl.ds(0, IDRAIN * BCH)], isem_a).wait()

    def start_gather(j, b):
        idx_slice = idx_v.at[pl.ds(pl.multiple_of(j * BCH, BCH), BCH)]
        pltpu.async_copy(emb_hbm.at[idx_slice], gbufs[b], gsems[b])

    for b in range(NBUF):
        start_gather(b, b)

    pltpu.make_async_copy(pos_hbm.at[pl.ds(0, PW)], pos_v, psem).wait()

    def do_chunk(j, b):
        @pl.when(j == IDRAIN - NBUF)
        def _():
            pltpu.make_async_copy(
                inp_hbm.at[pl.ds(0, (SW - IDRAIN) * BCH)],
                idx_v.at[pl.ds(0, (SW - IDRAIN) * BCH)], isem_b).wait()

        pltpu.make_async_copy(emb_hbm.at[pl.ds(0, BCH)], gbufs[b],
                              gsems[b]).wait()

        @pl.when(j >= NBUF)
        def _():
            pltpu.make_async_copy(
                obufs[b], out_hbm.at[pl.ds(0, BCH)], ssems[b]).wait()

        prow = (s0 + j) - align
        pvecs = [pos_v[prow, pl.ds(db * 16, 16)] for db in range(NV)]

        @plsc.parallel_loop(0, BCH, unroll=8)
        def _(bb, b=b, pvecs=pvecs):
            for db in range(NV):
                dsl = pl.ds(db * 16, 16)
                obufs[b][bb, dsl] = gbufs[b][bb, dsl] + pvecs[db]

        for g in range(NV):
            oaffs[b][pl.ds(g * 16, 16)] = (
                lanes_s + (base + g * 16 * S + j))
        pltpu.async_copy(obufs[b], out_hbm.at[oaffs[b]], ssems[b])

        @pl.when(j + NBUF < SW)
        def _():
            start_gather(j + NBUF, b)

    def outer(i, _):
        jo = i * NBUF
        for b in range(NBUF):
            do_chunk(jo + b, b)
        return 0

    nfull = SW // NBUF
    lax.fori_loop(0, nfull, outer, 0)
    for j in range(nfull * NBUF, SW):
        do_chunk(j, j % NBUF)

    for b in range(NBUF):
        pltpu.make_async_copy(obufs[b], out_hbm.at[pl.ds(0, BCH)],
                              ssems[b]).wait()


@functools.partial(jax.jit, static_argnums=())
def kernel(inp, embed_table, pos_table):
    mesh = plsc.VectorSubcoreMesh(core_axis_name="c", subcore_axis_name="s")
    scratch = (
        [pltpu.VMEM((SW * BCH,), jnp.int32),
         pltpu.VMEM((SW * BCH,), jnp.int32),
         pltpu.VMEM((S,), jnp.int32),
         pltpu.VMEM((PW, D), jnp.float32)]
        + [[pltpu.VMEM((BCH, D), jnp.float32) for _ in range(NBUF)]]
        + [[pltpu.VMEM((BCH, D), jnp.float32) for _ in range(NBUF)]]
        + [[pltpu.VMEM((BCH,), jnp.int32) for _ in range(NBUF)]]
        + [pltpu.SemaphoreType.DMA]
        + [pltpu.SemaphoreType.DMA]
        + [[pltpu.SemaphoreType.DMA for _ in range(NBUF)]]
        + [[pltpu.SemaphoreType.DMA for _ in range(NBUF)]]
        + [pltpu.SemaphoreType.DMA]
    )
    run = pl.kernel(
        _worker_body,
        out_type=jax.ShapeDtypeStruct((B * S, D), jnp.float32),
        mesh=mesh,
        scratch_types=scratch,
    )
    inp = inp.astype(jnp.int32)
    out = run(inp.reshape(B * S), embed_table, pos_table)
    return out.reshape(B, S, D)
